# baseline (speedup 1.0000x reference)
import functools

import jax
import jax.numpy as jnp
from jax import lax
from jax.experimental import pallas as pl
from jax.experimental.pallas import tpu as pltpu
from jax.experimental.pallas import tpu_sc as plsc

CHANNEL = 7
ROWS = 16384 * 100
LANES = 128
NUM_WORKERS = 32
ROWS_PER_WORKER = ROWS // NUM_WORKERS
VLEN = 16
NUM_VECS = ROWS_PER_WORKER // VLEN

_MESH = plsc.VectorSubcoreMesh(core_axis_name="c", subcore_axis_name="s")


@functools.partial(
    pl.kernel,
    mesh=_MESH,
    out_type=jax.ShapeDtypeStruct((ROWS,), jnp.float32),
    scratch_types=[
        pltpu.VMEM((ROWS_PER_WORKER,), jnp.int32),
        pltpu.VMEM((ROWS_PER_WORKER,), jnp.float32),
        pltpu.SemaphoreType.DMA,
    ],
    compiler_params=pltpu.CompilerParams(use_tc_tiling_on_sc=False),
)
def _channel_gather(x_hbm, out_hbm, idx_v, vals_v, sem):
    wid = lax.axis_index("s") * 2 + lax.axis_index("c")
    base = wid * ROWS_PER_WORKER
    lane = lax.iota(jnp.int32, 16)
    first = (base + lane) * LANES + CHANNEL

    def body(j, vec):
        idx_v[pl.ds(pl.multiple_of(j * VLEN, VLEN), VLEN)] = vec
        return vec + VLEN * LANES

    lax.fori_loop(0, NUM_VECS, body, first)

    pltpu.async_copy(x_hbm.at[idx_v], vals_v, sem).wait()
    pltpu.sync_copy(vals_v, out_hbm.at[pl.ds(base, ROWS_PER_WORKER)])


def kernel(inputs):
    x = inputs.reshape(ROWS * LANES)
    out = _channel_gather(x)
    return out.reshape(16384, 100, 1)


# speedup vs baseline: 15.8736x; 15.8736x over previous
import functools

import jax
import jax.numpy as jnp
from jax import lax
from jax.experimental import pallas as pl
from jax.experimental.pallas import tpu as pltpu
from jax.experimental.pallas import tpu_sc as plsc

CHANNEL = 7
ROWS = 16384 * 100
LANES = 128
NUM_WORKERS = 32
ROWS_PER_WORKER = ROWS // NUM_WORKERS
VLEN = 16
NUM_VECS = ROWS_PER_WORKER // VLEN

_MESH = plsc.VectorSubcoreMesh(core_axis_name="c", subcore_axis_name="s")


@functools.partial(
    pl.kernel,
    mesh=_MESH,
    out_type=jax.ShapeDtypeStruct((ROWS,), jnp.float32),
    scratch_types=[
        pltpu.VMEM((ROWS_PER_WORKER,), jnp.int32),
        pltpu.VMEM((ROWS_PER_WORKER,), jnp.float32),
        pltpu.SemaphoreType.DMA,
    ],
    compiler_params=pltpu.CompilerParams(use_tc_tiling_on_sc=False),
)
def _channel_gather(x_hbm, out_hbm, idx_v, vals_v, sem):
    wid = lax.axis_index("s") * 2 + lax.axis_index("c")
    base = wid * ROWS_PER_WORKER
    lane = lax.iota(jnp.int32, 16)
    first = (base + lane) * LANES + CHANNEL

    def body(j, vec):
        idx_v[pl.ds(pl.multiple_of(j * VLEN, VLEN), VLEN)] = vec
        return vec + VLEN * LANES

    lax.fori_loop(0, NUM_VECS, body, first)

    pltpu.async_copy(x_hbm.at[idx_v], vals_v, sem).wait()
    pltpu.sync_copy(vals_v, out_hbm.at[pl.ds(base, ROWS_PER_WORKER)])


def kernel(inputs):
    x = jnp.transpose(inputs, (1, 0, 2)).reshape(ROWS * LANES)
    out = _channel_gather(x)
    return out.reshape(100, 16384, 1).transpose(1, 0, 2)


# speedup vs baseline: 17.7992x; 1.1213x over previous
import functools

import jax
import jax.numpy as jnp
from jax import lax
from jax.experimental import pallas as pl
from jax.experimental.pallas import tpu as pltpu
from jax.experimental.pallas import tpu_sc as plsc

CHANNEL = 7
ROWS = 16384 * 100
LANES = 128
NUM_WORKERS = 32
ROWS_PER_WORKER = ROWS // NUM_WORKERS
VLEN = 16
NUM_CHUNKS = 4
CHUNK = ROWS_PER_WORKER // NUM_CHUNKS
VECS_PER_CHUNK = CHUNK // VLEN

_MESH = plsc.VectorSubcoreMesh(core_axis_name="c", subcore_axis_name="s")


@functools.partial(
    pl.kernel,
    mesh=_MESH,
    out_type=jax.ShapeDtypeStruct((ROWS,), jnp.float32),
    scratch_types=[
        pltpu.VMEM((ROWS_PER_WORKER,), jnp.int32),
        pltpu.VMEM((ROWS_PER_WORKER,), jnp.float32),
        pltpu.SemaphoreType.DMA,
        pltpu.SemaphoreType.DMA,
    ],
    compiler_params=pltpu.CompilerParams(use_tc_tiling_on_sc=False),
)
def _channel_gather(x_hbm, out_hbm, idx_v, vals_v, gsem, wsem):
    wid = lax.axis_index("s") * 2 + lax.axis_index("c")
    base = wid * ROWS_PER_WORKER
    lane = lax.iota(jnp.int32, 16)
    first = (base + lane) * LANES + CHANNEL

    def fill(c, vec):
        def body(j, v):
            idx_v[pl.ds(pl.multiple_of(c * CHUNK + j * VLEN, VLEN), VLEN)] = v
            return v + VLEN * LANES

        return lax.fori_loop(0, VECS_PER_CHUNK, body, vec)

    vec = first
    copies = []
    for c in range(NUM_CHUNKS):
        vec = fill(c, vec)
        cp = pltpu.make_async_copy(
            x_hbm.at[idx_v.at[pl.ds(c * CHUNK, CHUNK)]],
            vals_v.at[pl.ds(c * CHUNK, CHUNK)],
            gsem,
        )
        cp.start()
        copies.append(cp)

    wbs = []
    for c in range(NUM_CHUNKS):
        copies[c].wait()
        wb = pltpu.make_async_copy(
            vals_v.at[pl.ds(c * CHUNK, CHUNK)],
            out_hbm.at[pl.ds(base + c * CHUNK, CHUNK)],
            wsem,
        )
        wb.start()
        wbs.append(wb)
    for wb in wbs:
        wb.wait()


def kernel(inputs):
    x = jnp.transpose(inputs, (1, 0, 2)).reshape(ROWS * LANES)
    out = _channel_gather(x)
    return out.reshape(100, 16384, 1).transpose(1, 0, 2)
